# baseline (speedup 1.0000x reference)
import functools

import jax
import jax.numpy as jnp
from jax import lax
from jax.experimental import pallas as pl
from jax.experimental.pallas import tpu as pltpu
from jax.experimental.pallas import tpu_sc as plsc

ROWS = 1_000_000
COLS = 64
B = 16384
NC, NS, L = 2, 16, 16
NW = NC * NS
RPW = B // NW
CHUNK = 128
NCH = RPW // CHUNK

_mesh = plsc.VectorSubcoreMesh(core_axis_name="c", subcore_axis_name="s")


@functools.partial(
    pl.kernel,
    mesh=_mesh,
    out_type=jax.ShapeDtypeStruct((COLS, B), jnp.float32),
    scratch_types=[
        pltpu.VMEM((COLS, RPW), jnp.int32),
        pltpu.VMEM((COLS, RPW), jnp.float32),
        pltpu.SemaphoreType.DMA,
    ],
)
def _sc_gather(inp_flat, idx_t, out_t, idx_v, out_v, sem):
    wid = lax.axis_index("s") * NC + lax.axis_index("c")
    i0 = wid * RPW

    pltpu.sync_copy(idx_t.at[:, pl.ds(i0, RPW)], idx_v)

    def _copy(t):
        j = t // NCH
        c = t % NCH
        return pltpu.make_async_copy(
            inp_flat.at[pl.ds(j * ROWS, ROWS)].at[
                idx_v.at[j, pl.ds(c * CHUNK, CHUNK)]
            ],
            out_v.at[j, pl.ds(c * CHUNK, CHUNK)],
            sem,
        )

    def fire_body(t, _):
        _copy(t).start()
        return 0

    lax.fori_loop(0, COLS * NCH, fire_body, 0)

    def drain_body(t, _):
        _copy(t).wait()
        return 0

    lax.fori_loop(0, COLS * NCH, drain_body, 0)

    pltpu.sync_copy(out_v, out_t.at[:, pl.ds(i0, RPW)])


def kernel(input, index):
    flat = input.T.reshape(ROWS * COLS)
    out_t = _sc_gather(flat, index.T.astype(jnp.int32))
    return out_t.T


# speedup vs baseline: 7.7645x; 7.7645x over previous
import functools

import jax
import jax.numpy as jnp
from jax import lax
from jax.experimental import pallas as pl
from jax.experimental.pallas import tpu as pltpu
from jax.experimental.pallas import tpu_sc as plsc

ROWS = 1_000_000
COLS = 64
B = 16384
NC, NS, L = 2, 16, 16
NW = NC * NS
RPW = B // NW
CHUNK = 128
NCH = RPW // CHUNK

_mesh = plsc.VectorSubcoreMesh(core_axis_name="c", subcore_axis_name="s")


@functools.partial(
    pl.kernel,
    mesh=_mesh,
    out_type=jax.ShapeDtypeStruct((COLS, B), jnp.float32),
    scratch_types=[
        pltpu.VMEM((COLS, RPW), jnp.int32),
        pltpu.VMEM((COLS, RPW), jnp.float32),
        pltpu.SemaphoreType.DMA,
    ],
)
def _sc_gather(inp_flat, idx_t, out_t, idx_v, out_v, sem):
    wid = lax.axis_index("s") * NC + lax.axis_index("c")
    i0 = wid * RPW

    pltpu.sync_copy(idx_t.at[:, pl.ds(i0, RPW)], idx_v)

    def addr_body(j, _):
        for q in range(RPW // L):
            sl = pl.ds(q * L, L)
            idx_v[j, sl] = idx_v[j, sl] * COLS + j
        return 0

    lax.fori_loop(0, COLS, addr_body, 0)

    def _copy(t):
        j = t // NCH
        c = t % NCH
        return pltpu.make_async_copy(
            inp_flat.at[idx_v.at[j, pl.ds(c * CHUNK, CHUNK)]],
            out_v.at[j, pl.ds(c * CHUNK, CHUNK)],
            sem,
        )

    def fire_body(t, _):
        _copy(t).start()
        return 0

    lax.fori_loop(0, COLS * NCH, fire_body, 0)

    def drain_body(t, _):
        _copy(t).wait()
        return 0

    lax.fori_loop(0, COLS * NCH, drain_body, 0)

    pltpu.sync_copy(out_v, out_t.at[:, pl.ds(i0, RPW)])


def kernel(input, index):
    flat = input.reshape(ROWS * COLS)
    out_t = _sc_gather(flat, index.T.astype(jnp.int32))
    return out_t.T


# speedup vs baseline: 8.6243x; 1.1107x over previous
import functools

import jax
import jax.numpy as jnp
from jax import lax
from jax.experimental import pallas as pl
from jax.experimental.pallas import tpu as pltpu
from jax.experimental.pallas import tpu_sc as plsc

ROWS = 1_000_000
COLS = 64
B = 16384
NC, NS, L = 2, 16, 16
NW = NC * NS
RPW = B // NW
CHUNK = 128
NCH = RPW // CHUNK

_mesh = plsc.VectorSubcoreMesh(core_axis_name="c", subcore_axis_name="s")


@functools.partial(
    pl.kernel,
    mesh=_mesh,
    out_type=jax.ShapeDtypeStruct((COLS, B), jnp.float32),
    scratch_types=[
        pltpu.VMEM((COLS, RPW), jnp.int32),
        pltpu.VMEM((COLS, RPW), jnp.float32),
        pltpu.SemaphoreType.DMA,
    ],
)
def _sc_gather(inp_flat, idx_t, out_t, idx_v, out_v, sem):
    wid = lax.axis_index("s") * NC + lax.axis_index("c")
    i0 = wid * RPW

    pltpu.sync_copy(idx_t.at[:, pl.ds(i0, RPW)], idx_v)

    def addr_body(j, _):
        for q in range(RPW // L):
            sl = pl.ds(q * L, L)
            idx_v[j, sl] = idx_v[j, sl] * 128 + j
        return 0

    lax.fori_loop(0, COLS, addr_body, 0)

    def _copy(t):
        j = t // NCH
        c = t % NCH
        return pltpu.make_async_copy(
            inp_flat.at[idx_v.at[j, pl.ds(c * CHUNK, CHUNK)]],
            out_v.at[j, pl.ds(c * CHUNK, CHUNK)],
            sem,
        )

    def fire_body(t, _):
        _copy(t).start()
        return 0

    lax.fori_loop(0, COLS * NCH, fire_body, 0)

    def drain_body(t, _):
        _copy(t).wait()
        return 0

    lax.fori_loop(0, COLS * NCH, drain_body, 0)

    pltpu.sync_copy(out_v, out_t.at[:, pl.ds(i0, RPW)])


def kernel(input, index):
    padded = jnp.pad(input, ((0, 0), (0, 128 - COLS)))
    flat = padded.reshape(ROWS * 128)
    out_t = _sc_gather(flat, index.T.astype(jnp.int32))
    return out_t.T


# speedup vs baseline: 9.7784x; 1.1338x over previous
import functools

import jax
import jax.numpy as jnp
from jax import lax
from jax.experimental import pallas as pl
from jax.experimental.pallas import tpu as pltpu
from jax.experimental.pallas import tpu_sc as plsc

ROWS = 1_000_000
COLS = 64
B = 16384
NC, NS, L = 2, 16, 16
NW = NC * NS
RPW = B // NW
CHUNK = 128
NCH = RPW // CHUNK

_mesh = plsc.VectorSubcoreMesh(core_axis_name="c", subcore_axis_name="s")

_BLK = 2048


def _pad_xpose_body(x_ref, o_ref):
    xt = x_ref[...].T
    o_ref[...] = jnp.concatenate([xt, jnp.zeros_like(xt)], axis=1)


def _tc_pad_xpose(inp_t):
    n = (ROWS + _BLK - 1) // _BLK
    return pl.pallas_call(
        _pad_xpose_body,
        grid=(n,),
        in_specs=[pl.BlockSpec((COLS, _BLK), lambda i: (0, i))],
        out_specs=pl.BlockSpec((_BLK, 128), lambda i: (i, 0)),
        out_shape=jax.ShapeDtypeStruct((ROWS, 128), jnp.float32),
        compiler_params=pltpu.CompilerParams(
            dimension_semantics=("arbitrary",)
        ),
    )(inp_t)


@functools.partial(
    pl.kernel,
    mesh=_mesh,
    out_type=jax.ShapeDtypeStruct((COLS, B), jnp.float32),
    scratch_types=[
        pltpu.VMEM((COLS, RPW), jnp.int32),
        pltpu.VMEM((COLS, RPW), jnp.float32),
        pltpu.SemaphoreType.DMA,
    ],
)
def _sc_gather(inp_flat, idx_t, out_t, idx_v, out_v, sem):
    wid = lax.axis_index("s") * NC + lax.axis_index("c")
    i0 = wid * RPW

    pltpu.sync_copy(idx_t.at[:, pl.ds(i0, RPW)], idx_v)

    def addr_body(j, _):
        for q in range(RPW // L):
            sl = pl.ds(q * L, L)
            idx_v[j, sl] = idx_v[j, sl] * 128 + j
        return 0

    lax.fori_loop(0, COLS, addr_body, 0)

    def _copy(t):
        j = t // NCH
        c = t % NCH
        return pltpu.make_async_copy(
            inp_flat.at[idx_v.at[j, pl.ds(c * CHUNK, CHUNK)]],
            out_v.at[j, pl.ds(c * CHUNK, CHUNK)],
            sem,
        )

    def fire_body(t, _):
        _copy(t).start()
        return 0

    lax.fori_loop(0, COLS * NCH, fire_body, 0)

    def drain_body(t, _):
        _copy(t).wait()
        return 0

    lax.fori_loop(0, COLS * NCH, drain_body, 0)

    pltpu.sync_copy(out_v, out_t.at[:, pl.ds(i0, RPW)])


def kernel(input, index):
    padded = _tc_pad_xpose(input.T)
    flat = padded.reshape(ROWS * 128)
    out_t = _sc_gather(flat, index.T.astype(jnp.int32))
    return out_t.T
